# baseline (speedup 1.0000x reference)
import jax
import jax.numpy as jnp
from jax.experimental import pallas as pl
from jax.experimental.pallas import tpu as pltpu

BM = 400
NBUF = 3


def _gcn_kernel(adj_hbm, x_ref, w_ref, o_hbm, xw_ref, bufs, obuf,
                load_sems, store_sems):
    n = adj_hbm.shape[0]
    nblk = n // BM

    for b in range(NBUF):
        pltpu.make_async_copy(adj_hbm.at[pl.ds(b * BM, BM), :],
                              bufs.at[b], load_sems.at[b]).start()

    xw_ref[...] = jnp.dot(x_ref[...], w_ref[...],
                          preferred_element_type=jnp.float32)

    for i in range(nblk):
        s = i % NBUF
        t = i % 2
        pltpu.make_async_copy(adj_hbm.at[pl.ds(i * BM, BM), :],
                              bufs.at[s], load_sems.at[s]).wait()
        if i >= 2:
            pltpu.make_async_copy(obuf.at[t],
                                  o_hbm.at[pl.ds((i - 2) * BM, BM), :],
                                  store_sems.at[t]).wait()
        obuf[t] = jnp.tanh(jnp.dot(bufs[s], xw_ref[...],
                                   preferred_element_type=jnp.float32))
        pltpu.make_async_copy(obuf.at[t], o_hbm.at[pl.ds(i * BM, BM), :],
                              store_sems.at[t]).start()
        nxt = i + NBUF
        if nxt < nblk:
            pltpu.make_async_copy(adj_hbm.at[pl.ds(nxt * BM, BM), :],
                                  bufs.at[s], load_sems.at[s]).start()

    for i in (nblk - 2, nblk - 1):
        t = i % 2
        pltpu.make_async_copy(obuf.at[t], o_hbm.at[pl.ds(i * BM, BM), :],
                              store_sems.at[t]).wait()


def kernel(adj, x, W0):
    n, d_in = x.shape
    d_out = W0.shape[1]

    h = pl.pallas_call(
        _gcn_kernel,
        in_specs=[
            pl.BlockSpec(memory_space=pl.ANY),
            pl.BlockSpec(memory_space=pltpu.MemorySpace.VMEM),
            pl.BlockSpec(memory_space=pltpu.MemorySpace.VMEM),
        ],
        out_specs=pl.BlockSpec(memory_space=pl.ANY),
        out_shape=jax.ShapeDtypeStruct((n, d_out), jnp.float32),
        scratch_shapes=[
            pltpu.VMEM((n, d_out), jnp.float32),
            pltpu.VMEM((NBUF, BM, n), jnp.float32),
            pltpu.VMEM((2, BM, d_out), jnp.float32),
            pltpu.SemaphoreType.DMA((NBUF,)),
            pltpu.SemaphoreType.DMA((2,)),
        ],
    )(adj, x, W0)
    return h


# speedup vs baseline: 1.0343x; 1.0343x over previous
import jax
import jax.numpy as jnp
from jax.experimental import pallas as pl
from jax.experimental.pallas import tpu as pltpu


def _gcn_kernel(adj_ref, x_ref, w_ref, o_ref, xw_ref):
    @pl.when(pl.program_id(0) == 0)
    def _():
        xw_ref[...] = jnp.dot(x_ref[...], w_ref[...],
                              preferred_element_type=jnp.float32)

    acc = jnp.dot(adj_ref[...], xw_ref[...],
                  preferred_element_type=jnp.float32)
    o_ref[...] = jnp.tanh(acc)


def kernel(adj, x, W0):
    n, d_in = x.shape
    d_out = W0.shape[1]

    bm = 400
    h = pl.pallas_call(
        _gcn_kernel,
        grid=(n // bm,),
        in_specs=[
            pl.BlockSpec((bm, n), lambda i: (i, 0)),
            pl.BlockSpec((n, d_in), lambda i: (0, 0)),
            pl.BlockSpec((d_in, d_out), lambda i: (0, 0)),
        ],
        out_specs=pl.BlockSpec((bm, d_out), lambda i: (i, 0)),
        out_shape=jax.ShapeDtypeStruct((n, d_out), jnp.float32),
        scratch_shapes=[pltpu.VMEM((n, d_out), jnp.float32)],
        compiler_params=pltpu.CompilerParams(
            dimension_semantics=("arbitrary",),
        ),
    )(adj, x, W0)
    return h
